# baseline (speedup 1.0000x reference)
import functools
import jax
import jax.numpy as jnp
from jax import lax
from jax.experimental import pallas as pl
from jax.experimental.pallas import tpu as pltpu
from jax.experimental.pallas import tpu_sc as plsc

_F32 = jnp.float32
_I32 = jnp.int32

_NC = 2
_NS = 16
_NW = _NC * _NS
_L = 16


def _sc_mesh():
    return plsc.VectorSubcoreMesh(core_axis_name="c", subcore_axis_name="s",
                                  num_cores=_NC, num_subcores=_NS)


def _wid():
    return lax.axis_index("s") * _NC + lax.axis_index("c")


def _route_kernel(x_ref, wg_ref, dest_ref, gate_ref, off_ref, cnt_ref, *, E, CH):
    x = x_ref[...]
    wg = wg_ref[...]
    T = x.shape[0]
    logits = lax.dot_general(x, wg, (((1,), (1,)), ((), ())),
                             preferred_element_type=_F32)
    m = jnp.max(logits, axis=1, keepdims=True)
    p = jnp.exp(logits - m)
    rp = p / jnp.sum(p, axis=1, keepdims=True)
    gate = jnp.max(rp, axis=1, keepdims=True)
    iota_e = lax.broadcasted_iota(_I32, (T, E), 1)
    expert = jnp.min(jnp.where(rp == gate, iota_e, E), axis=1, keepdims=True)
    oh = (iota_e == expert).astype(_F32)

    counts = jnp.sum(oh, axis=0, keepdims=True)
    tri_e = (lax.broadcasted_iota(_I32, (E, E), 0)
             < lax.broadcasted_iota(_I32, (E, E), 1)).astype(_F32)
    offs = lax.dot_general(counts, tri_e, (((1,), (0,)), ((), ())),
                           precision=lax.Precision.HIGHEST,
                           preferred_element_type=_F32)
    off_ref[...] = offs.astype(_I32)
    cnt_ref[...] = counts.astype(_I32)
    gate_ref[...] = jnp.broadcast_to(gate, (T, 128))

    NCH = T // CH
    tri_c = (lax.broadcasted_iota(_I32, (CH, CH), 0)
             > lax.broadcasted_iota(_I32, (CH, CH), 1)).astype(_F32)
    tri_b = (lax.broadcasted_iota(_I32, (NCH, NCH), 0)
             > lax.broadcasted_iota(_I32, (NCH, NCH), 1)).astype(_F32)
    csums = []
    for c in range(NCH):
        ohc = oh[c * CH:(c + 1) * CH, :]
        csums.append(jnp.sum(ohc, axis=0, keepdims=True))
    chunk_sums = jnp.concatenate(csums, axis=0)
    bases = lax.dot_general(tri_b, chunk_sums, (((1,), (0,)), ((), ())),
                            precision=lax.Precision.HIGHEST,
                            preferred_element_type=_F32)
    for c in range(NCH):
        ohc = oh[c * CH:(c + 1) * CH, :]
        rank = lax.dot_general(tri_c, ohc, (((1,), (0,)), ((), ())),
                               preferred_element_type=_F32)
        rank = rank + (bases[c:c + 1, :] + offs)
        dest = jnp.sum(ohc * rank, axis=1, keepdims=True)
        dest_ref[pl.ds(c * CH, CH), :] = dest.astype(_I32)


def _route(flat, Wg):
    T, D = flat.shape
    E = Wg.shape[0]
    CH = 128
    dest, gate128, offs, cnts = pl.pallas_call(
        functools.partial(_route_kernel, E=E, CH=CH),
        out_shape=[
            jax.ShapeDtypeStruct((T, 1), _I32),
            jax.ShapeDtypeStruct((T, 128), _F32),
            jax.ShapeDtypeStruct((1, E), _I32),
            jax.ShapeDtypeStruct((1, E), _I32),
        ],
    )(flat, Wg)
    return dest, gate128, offs, cnts


def _sc_gather_fwd(dest, flat, gate128):
    T, D = flat.shape
    G = gate128.shape[1]
    bpw = T // _NW

    @functools.partial(
        pl.kernel,
        mesh=_sc_mesh(),
        out_type=[
            jax.ShapeDtypeStruct((T, D), _F32),
            jax.ShapeDtypeStruct((T, G), _F32),
        ],
        scratch_types=[
            pltpu.VMEM((T,), _I32),
            pltpu.VMEM((bpw,), _I32),
            pltpu.VMEM((bpw, D), _F32),
            pltpu.VMEM((bpw, G), _F32),
            pltpu.SemaphoreType.DMA,
        ],
        compiler_params=pltpu.CompilerParams(needs_layout_passes=False),
    )
    def k(dest_hbm, flat_hbm, gate_hbm, xs_hbm, gs_hbm,
          dest_v, idx_v, rows_v, grows_v, sem):
        base = _wid() * bpw
        pltpu.sync_copy(dest_hbm, dest_v)
        lane = lax.iota(_I32, _L)

        def body(j, carry):
            rel = dest_v[pl.ds(j * _L, _L)] - base
            m = (rel >= 0) & (rel < bpw)
            plsc.store_scatter(idx_v, [rel], j * _L + lane, mask=m)
            return carry

        lax.fori_loop(0, T // _L, body, 0)
        pltpu.async_copy(flat_hbm.at[idx_v], rows_v, sem).wait()
        pltpu.sync_copy(rows_v, xs_hbm.at[pl.ds(base, bpw)])
        pltpu.async_copy(gate_hbm.at[idx_v], grows_v, sem).wait()
        pltpu.sync_copy(grows_v, gs_hbm.at[pl.ds(base, bpw)])

    return k(dest, flat, gate128)


def _sc_unsort(dest, y_sorted):
    T, D = y_sorted.shape
    bpw = T // _NW

    @functools.partial(
        pl.kernel,
        mesh=_sc_mesh(),
        out_type=jax.ShapeDtypeStruct((T, D), _F32),
        scratch_types=[
            pltpu.VMEM((bpw,), _I32),
            pltpu.VMEM((bpw, D), _F32),
            pltpu.SemaphoreType.DMA,
        ],
    )
    def k(dest_hbm, y_hbm, out_hbm, idx_v, rows_v, sem):
        base = _wid() * bpw
        pltpu.sync_copy(dest_hbm.at[pl.ds(base, bpw)], idx_v)
        pltpu.async_copy(y_hbm.at[idx_v], rows_v, sem).wait()
        pltpu.sync_copy(rows_v, out_hbm.at[pl.ds(base, bpw)])

    return k(dest, y_sorted)


def _mm_kernel(off_ref, cnt_ref, x_ref, gs_ref, wup_ref, bup_ref,
               wdown_ref, bdown_ref, out_ref, *, BM, NF):
    e = pl.program_id(0)
    f = pl.program_id(1)

    @pl.when((e == 0) & (f == 0))
    def _init():
        out_ref[...] = jnp.zeros_like(out_ref)

    off = off_ref[e]
    cnt = cnt_ref[e]
    fb = off // BM
    nb = (off + cnt + (BM - 1)) // BM - fb

    wup = wup_ref[0]
    wdn = wdown_ref[0]
    bu = bup_ref[0]
    bd = bdown_ref[0]

    def body(k, carry):
        lo = (fb + k) * BM
        xa = x_ref[pl.ds(lo, BM), :]
        hT = lax.dot_general(wup, xa, (((1,), (1,)), ((), ())),
                             preferred_element_type=_F32)
        hT = hT + bu
        hT = 0.5 * hT * (1.0 + lax.erf(hT * 0.7071067811865476))
        yT = lax.dot_general(wdn, hT, (((1,), (0,)), ((), ())),
                             preferred_element_type=_F32)
        yT = yT + jnp.where(f == 0, 1.0, 0.0) * bd
        y = lax.transpose(yT, (1, 0))
        y = y * gs_ref[pl.ds(lo, BM), :1]
        g_row = lo + lax.broadcasted_iota(_I32, (BM, 1), 0)
        mask = (g_row >= off) & (g_row < off + cnt)
        prev = out_ref[pl.ds(lo, BM), :]
        out_ref[pl.ds(lo, BM), :] = jnp.where(mask, prev + y, prev)
        return carry

    jax.lax.fori_loop(0, nb, body, 0)


def _grouped_ffn(x_sorted, gate_sorted, Wup, bup, Wdown, bdown, offs, cnts,
                 BM=256, FFB=1536):
    T, D = x_sorted.shape
    E, FF, _ = Wup.shape
    NF = FF // FFB
    grid = (E, NF)
    return pl.pallas_call(
        functools.partial(_mm_kernel, BM=BM, NF=NF),
        grid=grid,
        in_specs=[
            pl.BlockSpec(memory_space=pltpu.SMEM),
            pl.BlockSpec(memory_space=pltpu.SMEM),
            pl.BlockSpec((T, D), lambda e, f: (0, 0)),
            pl.BlockSpec((T, 128), lambda e, f: (0, 0)),
            pl.BlockSpec((1, FFB, D), lambda e, f: (e, f, 0)),
            pl.BlockSpec((1, FFB, 1), lambda e, f: (e * NF + f, 0, 0)),
            pl.BlockSpec((1, D, FFB), lambda e, f: (e, 0, f)),
            pl.BlockSpec((1, D, 1), lambda e, f: (e, 0, 0)),
        ],
        out_specs=pl.BlockSpec((T, D), lambda e, f: (0, 0)),
        out_shape=jax.ShapeDtypeStruct((T, D), _F32),
        compiler_params=pltpu.CompilerParams(
            dimension_semantics=("arbitrary", "arbitrary"),
        ),
    )(offs, cnts, x_sorted, gate_sorted,
      Wup, bup.reshape(E * NF, FFB, 1), Wdown, bdown.reshape(E, D, 1))


def kernel(hidden_states, Wg, Wup, bup, Wdown, bdown):
    B, S, D = hidden_states.shape
    E = Wg.shape[0]
    flat = hidden_states.reshape(-1, D)
    T = flat.shape[0]

    dest, gate128, offs, cnts = _route(flat, Wg)
    dest1 = dest.reshape(T)
    offs1 = offs.reshape(E)
    cnts1 = cnts.reshape(E)

    x_sorted, gate_sorted = _sc_gather_fwd(dest1, flat, gate128)
    y_sorted = _grouped_ffn(x_sorted, gate_sorted, Wup, bup, Wdown, bdown,
                            offs1, cnts1)
    out = _sc_unsort(dest1, y_sorted)
    return out.reshape(B, S, D)


# speedup vs baseline: 1.3926x; 1.3926x over previous
import functools
import jax
import jax.numpy as jnp
from jax import lax
from jax.experimental import pallas as pl
from jax.experimental.pallas import tpu as pltpu
from jax.experimental.pallas import tpu_sc as plsc

_F32 = jnp.float32
_I32 = jnp.int32

_NC = 2
_NS = 16
_NW = _NC * _NS
_L = 16


def _sc_mesh():
    return plsc.VectorSubcoreMesh(core_axis_name="c", subcore_axis_name="s",
                                  num_cores=_NC, num_subcores=_NS)


def _wid():
    return lax.axis_index("s") * _NC + lax.axis_index("c")


def _route_kernel(x_ref, wg_ref, dest_ref, gate_ref, off_ref, cnt_ref, *, E, CH):
    x = x_ref[...]
    wg = wg_ref[...]
    T = x.shape[0]
    logits = lax.dot_general(x, wg, (((1,), (1,)), ((), ())),
                             preferred_element_type=_F32)
    m = jnp.max(logits, axis=1, keepdims=True)
    p = jnp.exp(logits - m)
    rp = p / jnp.sum(p, axis=1, keepdims=True)
    gate = jnp.max(rp, axis=1, keepdims=True)
    iota_e = lax.broadcasted_iota(_I32, (T, E), 1)
    expert = jnp.min(jnp.where(rp == gate, iota_e, E), axis=1, keepdims=True)
    oh = (iota_e == expert).astype(_F32)

    counts = jnp.sum(oh, axis=0, keepdims=True)
    tri_e = (lax.broadcasted_iota(_I32, (E, E), 0)
             < lax.broadcasted_iota(_I32, (E, E), 1)).astype(_F32)
    offs = lax.dot_general(counts, tri_e, (((1,), (0,)), ((), ())),
                           precision=lax.Precision.HIGHEST,
                           preferred_element_type=_F32)
    off_ref[...] = offs.astype(_I32)
    cnt_ref[...] = counts.astype(_I32)
    gate_ref[...] = jnp.broadcast_to(gate, (T, 128))

    NCH = T // CH
    tri_c = (lax.broadcasted_iota(_I32, (CH, CH), 0)
             > lax.broadcasted_iota(_I32, (CH, CH), 1)).astype(_F32)
    tri_b = (lax.broadcasted_iota(_I32, (NCH, NCH), 0)
             > lax.broadcasted_iota(_I32, (NCH, NCH), 1)).astype(_F32)
    csums = []
    for c in range(NCH):
        ohc = oh[c * CH:(c + 1) * CH, :]
        csums.append(jnp.sum(ohc, axis=0, keepdims=True))
    chunk_sums = jnp.concatenate(csums, axis=0)
    bases = lax.dot_general(tri_b, chunk_sums, (((1,), (0,)), ((), ())),
                            precision=lax.Precision.HIGHEST,
                            preferred_element_type=_F32)
    for c in range(NCH):
        ohc = oh[c * CH:(c + 1) * CH, :]
        rank = lax.dot_general(tri_c, ohc, (((1,), (0,)), ((), ())),
                               preferred_element_type=_F32)
        rank = rank + (bases[c:c + 1, :] + offs)
        dest = jnp.sum(ohc * rank, axis=1, keepdims=True)
        dest_ref[pl.ds(c * CH, CH), :] = dest.astype(_I32)


def _route(flat, Wg):
    T, D = flat.shape
    E = Wg.shape[0]
    CH = 128
    dest, gate128, offs, cnts = pl.pallas_call(
        functools.partial(_route_kernel, E=E, CH=CH),
        out_shape=[
            jax.ShapeDtypeStruct((T, 1), _I32),
            jax.ShapeDtypeStruct((T, 128), _F32),
            jax.ShapeDtypeStruct((1, E), _I32),
            jax.ShapeDtypeStruct((1, E), _I32),
        ],
    )(flat, Wg)
    return dest, gate128, offs, cnts


def _sc_gather_fwd(dest, flat, gate128):
    T, D = flat.shape
    G = gate128.shape[1]
    bpw = T // _NW

    @functools.partial(
        pl.kernel,
        mesh=_sc_mesh(),
        out_type=[
            jax.ShapeDtypeStruct((T, D), _F32),
            jax.ShapeDtypeStruct((T, G), _F32),
        ],
        scratch_types=[
            pltpu.VMEM((T,), _I32),
            pltpu.VMEM((bpw,), _I32),
            pltpu.VMEM((bpw, D), _F32),
            pltpu.VMEM((bpw, G), _F32),
            pltpu.SemaphoreType.DMA,
        ],
        compiler_params=pltpu.CompilerParams(needs_layout_passes=False),
    )
    def k(dest_hbm, flat_hbm, gate_hbm, xs_hbm, gs_hbm,
          dest_v, idx_v, rows_v, grows_v, sem):
        base = _wid() * bpw
        pltpu.sync_copy(dest_hbm, dest_v)
        lane = lax.iota(_I32, _L)

        def body(j, carry):
            rel = dest_v[pl.ds(j * _L, _L)] - base
            m = (rel >= 0) & (rel < bpw)
            plsc.store_scatter(idx_v, [rel], j * _L + lane, mask=m)
            return carry

        lax.fori_loop(0, T // _L, body, 0)
        pltpu.async_copy(flat_hbm.at[idx_v], rows_v, sem).wait()
        pltpu.sync_copy(rows_v, xs_hbm.at[pl.ds(base, bpw)])
        pltpu.async_copy(gate_hbm.at[idx_v], grows_v, sem).wait()
        pltpu.sync_copy(grows_v, gs_hbm.at[pl.ds(base, bpw)])

    return k(dest, flat, gate128)


def _sc_unsort(dest, y_sorted):
    T, D = y_sorted.shape
    bpw = T // _NW

    @functools.partial(
        pl.kernel,
        mesh=_sc_mesh(),
        out_type=jax.ShapeDtypeStruct((T, D), _F32),
        scratch_types=[
            pltpu.VMEM((bpw,), _I32),
            pltpu.VMEM((bpw, D), _F32),
            pltpu.SemaphoreType.DMA,
        ],
    )
    def k(dest_hbm, y_hbm, out_hbm, idx_v, rows_v, sem):
        base = _wid() * bpw
        pltpu.sync_copy(dest_hbm.at[pl.ds(base, bpw)], idx_v)
        pltpu.async_copy(y_hbm.at[idx_v], rows_v, sem).wait()
        pltpu.sync_copy(rows_v, out_hbm.at[pl.ds(base, bpw)])

    return k(dest, y_sorted)


def _mm_kernel(off_ref, cnt_ref, x_ref, gs_ref, wup_ref, bup_ref,
               wdown_ref, bdown_ref, out_ref, *, BM, NF):
    e = pl.program_id(0)
    f = pl.program_id(1)

    @pl.when((e == 0) & (f == 0))
    def _init():
        out_ref[...] = jnp.zeros_like(out_ref)

    off = off_ref[e]
    cnt = cnt_ref[e]
    fb = off // BM
    nb = (off + cnt + (BM - 1)) // BM - fb

    wup = wup_ref[0]
    wdn = wdown_ref[0]
    bu = bup_ref[0]
    bd = bdown_ref[0]

    def body(k, carry):
        lo = (fb + k) * BM
        xa = x_ref[pl.ds(lo, BM), :]
        h = lax.dot_general(xa, wup, (((1,), (1,)), ((), ())),
                            preferred_element_type=_F32)
        h = h + bu
        h = 0.5 * h * (1.0 + lax.erf(h * 0.7071067811865476))
        y = lax.dot_general(h, wdn, (((1,), (1,)), ((), ())),
                            preferred_element_type=_F32)
        y = y + jnp.where(f == 0, 1.0, 0.0) * bd
        y = y * gs_ref[pl.ds(lo, BM), :1]
        g_row = lo + lax.broadcasted_iota(_I32, (BM, 1), 0)
        mask = (g_row >= off) & (g_row < off + cnt)
        prev = out_ref[pl.ds(lo, BM), :]
        out_ref[pl.ds(lo, BM), :] = jnp.where(mask, prev + y, prev)
        return carry

    jax.lax.fori_loop(0, nb, body, 0)


def _grouped_ffn(x_sorted, gate_sorted, Wup, bup, Wdown, bdown, offs, cnts,
                 BM=256, FFB=1536):
    T, D = x_sorted.shape
    E, FF, _ = Wup.shape
    NF = FF // FFB
    grid = (E, NF)
    return pl.pallas_call(
        functools.partial(_mm_kernel, BM=BM, NF=NF),
        grid=grid,
        in_specs=[
            pl.BlockSpec(memory_space=pltpu.SMEM),
            pl.BlockSpec(memory_space=pltpu.SMEM),
            pl.BlockSpec((T, D), lambda e, f: (0, 0)),
            pl.BlockSpec((T, 128), lambda e, f: (0, 0)),
            pl.BlockSpec((1, FFB, D), lambda e, f: (e, f, 0)),
            pl.BlockSpec((1, 1, FFB), lambda e, f: (e * NF + f, 0, 0)),
            pl.BlockSpec((1, D, FFB), lambda e, f: (e, 0, f)),
            pl.BlockSpec((1, 1, D), lambda e, f: (e, 0, 0)),
        ],
        out_specs=pl.BlockSpec((T, D), lambda e, f: (0, 0)),
        out_shape=jax.ShapeDtypeStruct((T, D), _F32),
        compiler_params=pltpu.CompilerParams(
            dimension_semantics=("arbitrary", "arbitrary"),
        ),
    )(offs, cnts, x_sorted, gate_sorted,
      Wup, bup.reshape(E * NF, 1, FFB), Wdown, bdown.reshape(E, 1, D))


def kernel(hidden_states, Wg, Wup, bup, Wdown, bdown):
    B, S, D = hidden_states.shape
    E = Wg.shape[0]
    flat = hidden_states.reshape(-1, D)
    T = flat.shape[0]

    dest, gate128, offs, cnts = _route(flat, Wg)
    dest1 = dest.reshape(T)
    offs1 = offs.reshape(E)
    cnts1 = cnts.reshape(E)

    x_sorted, gate_sorted = _sc_gather_fwd(dest1, flat, gate128)
    y_sorted = _grouped_ffn(x_sorted, gate_sorted, Wup, bup, Wdown, bdown,
                            offs1, cnts1)
    out = _sc_unsort(dest1, y_sorted)
    return out.reshape(B, S, D)


# speedup vs baseline: 1.4364x; 1.0315x over previous
import functools
import jax
import jax.numpy as jnp
from jax import lax
from jax.experimental import pallas as pl
from jax.experimental.pallas import tpu as pltpu
from jax.experimental.pallas import tpu_sc as plsc

_F32 = jnp.float32
_I32 = jnp.int32

_NC = 2
_NS = 16
_NW = _NC * _NS
_L = 16


def _sc_mesh():
    return plsc.VectorSubcoreMesh(core_axis_name="c", subcore_axis_name="s",
                                  num_cores=_NC, num_subcores=_NS)


def _wid():
    return lax.axis_index("s") * _NC + lax.axis_index("c")


def _route_kernel(x_ref, wg_ref, dest_ref, gate_ref, off_ref, cnt_ref, *, E, CH):
    x = x_ref[...]
    wg = wg_ref[...]
    T = x.shape[0]
    logits = lax.dot_general(x, wg, (((1,), (1,)), ((), ())),
                             preferred_element_type=_F32)
    m = jnp.max(logits, axis=1, keepdims=True)
    p = jnp.exp(logits - m)
    rp = p / jnp.sum(p, axis=1, keepdims=True)
    gate = jnp.max(rp, axis=1, keepdims=True)
    iota_e = lax.broadcasted_iota(_I32, (T, E), 1)
    expert = jnp.min(jnp.where(rp == gate, iota_e, E), axis=1, keepdims=True)
    oh = (iota_e == expert).astype(_F32)

    counts = jnp.sum(oh, axis=0, keepdims=True)
    tri_e = (lax.broadcasted_iota(_I32, (E, E), 0)
             < lax.broadcasted_iota(_I32, (E, E), 1)).astype(_F32)
    offs = lax.dot_general(counts, tri_e, (((1,), (0,)), ((), ())),
                           precision=lax.Precision.HIGHEST,
                           preferred_element_type=_F32)
    off_ref[...] = offs.astype(_I32)
    cnt_ref[...] = counts.astype(_I32)
    gate_ref[...] = jnp.broadcast_to(gate, (T, 128))

    NCH = T // CH
    tri_c = (lax.broadcasted_iota(_I32, (CH, CH), 0)
             > lax.broadcasted_iota(_I32, (CH, CH), 1)).astype(_F32)
    tri_b = (lax.broadcasted_iota(_I32, (NCH, NCH), 0)
             > lax.broadcasted_iota(_I32, (NCH, NCH), 1)).astype(_F32)
    csums = []
    for c in range(NCH):
        ohc = oh[c * CH:(c + 1) * CH, :]
        csums.append(jnp.sum(ohc, axis=0, keepdims=True))
    chunk_sums = jnp.concatenate(csums, axis=0)
    bases = lax.dot_general(tri_b, chunk_sums, (((1,), (0,)), ((), ())),
                            precision=lax.Precision.HIGHEST,
                            preferred_element_type=_F32)
    for c in range(NCH):
        ohc = oh[c * CH:(c + 1) * CH, :]
        rank = lax.dot_general(tri_c, ohc, (((1,), (0,)), ((), ())),
                               preferred_element_type=_F32)
        rank = rank + (bases[c:c + 1, :] + offs)
        dest = jnp.sum(ohc * rank, axis=1, keepdims=True)
        dest_ref[pl.ds(c * CH, CH), :] = dest.astype(_I32)


def _route(flat, Wg):
    T, D = flat.shape
    E = Wg.shape[0]
    CH = 128
    dest, gate128, offs, cnts = pl.pallas_call(
        functools.partial(_route_kernel, E=E, CH=CH),
        out_shape=[
            jax.ShapeDtypeStruct((T, 1), _I32),
            jax.ShapeDtypeStruct((T, 128), _F32),
            jax.ShapeDtypeStruct((1, E), _I32),
            jax.ShapeDtypeStruct((1, E), _I32),
        ],
    )(flat, Wg)
    return dest, gate128, offs, cnts


def _sc_gather_fwd(dest, flat, gate128):
    T, D = flat.shape
    G = gate128.shape[1]
    bpw = T // _NW

    @functools.partial(
        pl.kernel,
        mesh=_sc_mesh(),
        out_type=[
            jax.ShapeDtypeStruct((T, D), _F32),
            jax.ShapeDtypeStruct((T, G), _F32),
        ],
        scratch_types=[
            pltpu.VMEM((T,), _I32),
            pltpu.VMEM((bpw,), _I32),
            pltpu.VMEM((bpw, D), _F32),
            pltpu.VMEM((bpw, G), _F32),
            pltpu.SemaphoreType.DMA,
        ],
        compiler_params=pltpu.CompilerParams(needs_layout_passes=False),
    )
    def k(dest_hbm, flat_hbm, gate_hbm, xs_hbm, gs_hbm,
          dest_v, idx_v, rows_v, grows_v, sem):
        base = _wid() * bpw
        pltpu.sync_copy(dest_hbm, dest_v)
        lane = lax.iota(_I32, _L)

        def body(j, carry):
            rel = dest_v[pl.ds(j * _L, _L)] - base
            m = (rel >= 0) & (rel < bpw)
            plsc.store_scatter(idx_v, [rel], j * _L + lane, mask=m)
            return carry

        lax.fori_loop(0, T // _L, body, 0)
        pltpu.async_copy(flat_hbm.at[idx_v], rows_v, sem).wait()
        pltpu.sync_copy(rows_v, xs_hbm.at[pl.ds(base, bpw)])
        pltpu.async_copy(gate_hbm.at[idx_v], grows_v, sem).wait()
        pltpu.sync_copy(grows_v, gs_hbm.at[pl.ds(base, bpw)])

    return k(dest, flat, gate128)


def _sc_unsort(dest, y_sorted):
    T, D = y_sorted.shape
    bpw = T // _NW

    @functools.partial(
        pl.kernel,
        mesh=_sc_mesh(),
        out_type=jax.ShapeDtypeStruct((T, D), _F32),
        scratch_types=[
            pltpu.VMEM((bpw,), _I32),
            pltpu.VMEM((bpw, D), _F32),
            pltpu.SemaphoreType.DMA,
        ],
    )
    def k(dest_hbm, y_hbm, out_hbm, idx_v, rows_v, sem):
        base = _wid() * bpw
        pltpu.sync_copy(dest_hbm.at[pl.ds(base, bpw)], idx_v)
        pltpu.async_copy(y_hbm.at[idx_v], rows_v, sem).wait()
        pltpu.sync_copy(rows_v, out_hbm.at[pl.ds(base, bpw)])

    return k(dest, y_sorted)


def _mm_kernel(off_ref, cnt_ref, x_ref, gs_ref, wup_ref, bup_ref,
               wdown_ref, bdown_ref, out_ref, *, BM, NF):
    e = pl.program_id(0)
    f = pl.program_id(1)

    @pl.when((e == 0) & (f == 0))
    def _init():
        out_ref[...] = jnp.zeros_like(out_ref)

    off = off_ref[e]
    cnt = cnt_ref[e]
    AL = BM // 2
    lo0 = (off // AL) * AL
    nb = (off + cnt - lo0 + (BM - 1)) // BM

    wup = wup_ref[0]
    wdn = wdown_ref[0]
    bu = bup_ref[0]
    bd = bdown_ref[0]

    def body(k, carry):
        lo_u = lo0 + k * BM
        lo = jnp.minimum(lo_u, out_ref.shape[0] - BM)
        xa = x_ref[pl.ds(lo, BM), :]
        h = lax.dot_general(xa, wup, (((1,), (1,)), ((), ())),
                            preferred_element_type=_F32)
        h = h + bu
        h = 0.5 * h * (1.0 + lax.erf(h * 0.7071067811865476))
        y = lax.dot_general(h, wdn, (((1,), (1,)), ((), ())),
                            preferred_element_type=_F32)
        y = y + jnp.where(f == 0, 1.0, 0.0) * bd
        y = y * gs_ref[pl.ds(lo, BM), :1]
        g_row = lo + lax.broadcasted_iota(_I32, (BM, 1), 0)
        mask = (g_row >= off) & (g_row < off + cnt) & (g_row >= lo_u)
        prev = out_ref[pl.ds(lo, BM), :]
        out_ref[pl.ds(lo, BM), :] = jnp.where(mask, prev + y, prev)
        return carry

    jax.lax.fori_loop(0, nb, body, 0)


def _grouped_ffn(x_sorted, gate_sorted, Wup, bup, Wdown, bdown, offs, cnts,
                 BM=256, FFB=1536):
    T, D = x_sorted.shape
    E, FF, _ = Wup.shape
    NF = FF // FFB
    grid = (E, NF)
    return pl.pallas_call(
        functools.partial(_mm_kernel, BM=BM, NF=NF),
        grid=grid,
        in_specs=[
            pl.BlockSpec(memory_space=pltpu.SMEM),
            pl.BlockSpec(memory_space=pltpu.SMEM),
            pl.BlockSpec((T, D), lambda e, f: (0, 0)),
            pl.BlockSpec((T, 128), lambda e, f: (0, 0)),
            pl.BlockSpec((1, FFB, D), lambda e, f: (e, f, 0)),
            pl.BlockSpec((1, 1, FFB), lambda e, f: (e * NF + f, 0, 0)),
            pl.BlockSpec((1, D, FFB), lambda e, f: (e, 0, f)),
            pl.BlockSpec((1, 1, D), lambda e, f: (e, 0, 0)),
        ],
        out_specs=pl.BlockSpec((T, D), lambda e, f: (0, 0)),
        out_shape=jax.ShapeDtypeStruct((T, D), _F32),
        compiler_params=pltpu.CompilerParams(
            dimension_semantics=("arbitrary", "arbitrary"),
        ),
    )(offs, cnts, x_sorted, gate_sorted,
      Wup, bup.reshape(E * NF, 1, FFB), Wdown, bdown.reshape(E, 1, D))


def kernel(hidden_states, Wg, Wup, bup, Wdown, bdown):
    B, S, D = hidden_states.shape
    E = Wg.shape[0]
    flat = hidden_states.reshape(-1, D)
    T = flat.shape[0]

    dest, gate128, offs, cnts = _route(flat, Wg)
    dest1 = dest.reshape(T)
    offs1 = offs.reshape(E)
    cnts1 = cnts.reshape(E)

    x_sorted, gate_sorted = _sc_gather_fwd(dest1, flat, gate128)
    y_sorted = _grouped_ffn(x_sorted, gate_sorted, Wup, bup, Wdown, bdown,
                            offs1, cnts1)
    out = _sc_unsort(dest1, y_sorted)
    return out.reshape(B, S, D)
